# baseline (speedup 1.0000x reference)
import functools

import jax
import jax.numpy as jnp
from jax import lax
from jax.experimental import pallas as pl
from jax.experimental.pallas import tpu as pltpu
from jax.experimental.pallas import tpu_sc as plsc

NUM_E = 1000000
NUM_R = 1000
D = 64
B = 16384

NC = 2
NS = 16
NW = NC * NS
RPT = B // NW


def _sc_gather(s, r, o, sp, op, e_table, r_table):
    mesh = plsc.VectorSubcoreMesh(core_axis_name="c", subcore_axis_name="s")

    @functools.partial(
        pl.kernel,
        out_type=jax.ShapeDtypeStruct((5 * B, D), jnp.float32),
        mesh=mesh,
        scratch_types=[
            pltpu.VMEM((RPT,), jnp.int32),
            pltpu.VMEM((RPT, D), jnp.float32),
            pltpu.SemaphoreType.DMA,
            pltpu.SemaphoreType.DMA,
        ],
        compiler_params=pltpu.CompilerParams(use_tc_tiling_on_sc=True),
    )
    def k(s_h, r_h, o_h, sp_h, op_h, e_h, rel_h, out_h, idx_v, rows_v,
          isem, gsem):
        wid = lax.axis_index("s") * NC + lax.axis_index("c")
        base = wid * RPT
        for slot, (idx_h, tbl_h) in enumerate(
            [(s_h, e_h), (r_h, rel_h), (o_h, e_h), (sp_h, e_h), (op_h, e_h)]
        ):
            pltpu.async_copy(idx_h.at[pl.ds(base, RPT)], idx_v, isem).wait()

            @pl.loop(0, RPT, step=16)
            def _fire(g):
                vec = idx_v[pl.ds(g, 16)]
                for l in range(16):
                    pltpu.async_copy(
                        tbl_h.at[pl.ds(vec[l], 1)], rows_v.at[pl.ds(g + l, 1)],
                        gsem,
                    )

            @pl.loop(0, RPT)
            def _drain(j):
                pltpu.make_async_copy(
                    tbl_h.at[pl.ds(0, 1)], rows_v.at[pl.ds(j, 1)], gsem
                ).wait()

            pltpu.sync_copy(rows_v, out_h.at[pl.ds(slot * B + base, RPT)])

    return k(s, r, o, sp, op, e_table, r_table)


def _tc_normalize(rows):
    n_rows = rows.shape[0]
    blk = 8192

    def body(x_ref, o_ref):
        x = x_ref[...]
        ss = jnp.sum(x * x, axis=1, keepdims=True)
        norm = jnp.sqrt(ss)
        o_ref[...] = x / jnp.maximum(norm, 1e-12)

    return pl.pallas_call(
        body,
        out_shape=jax.ShapeDtypeStruct((n_rows, D), jnp.float32),
        grid=(n_rows // blk,),
        in_specs=[pl.BlockSpec((blk, D), lambda i: (i, 0))],
        out_specs=pl.BlockSpec((blk, D), lambda i: (i, 0)),
    )(rows)


def kernel(s, r, o, sp, op, e_table, r_table):
    s = s.astype(jnp.int32)
    r = r.astype(jnp.int32)
    o = o.astype(jnp.int32)
    sp = sp.astype(jnp.int32)
    op = op.astype(jnp.int32)
    rows = _sc_gather(s, r, o, sp, op, e_table, r_table)
    out = _tc_normalize(rows)
    return out.reshape(5, B, D)


# speedup vs baseline: 1.1924x; 1.1924x over previous
import functools

import jax
import jax.numpy as jnp
from jax import lax
from jax.experimental import pallas as pl
from jax.experimental.pallas import tpu as pltpu
from jax.experimental.pallas import tpu_sc as plsc

NUM_E = 1000000
NUM_R = 1000
D = 64
B = 16384

NC = 2
NS = 16
NW = NC * NS
RPT = B // NW


def _tc_transpose(xt):
    n = xt.shape[1]
    blk = 8192

    def body(x_ref, o_ref):
        o_ref[...] = x_ref[...].T

    return pl.pallas_call(
        body,
        out_shape=jax.ShapeDtypeStruct((n, D), jnp.float32),
        grid=(pl.cdiv(n, blk),),
        in_specs=[pl.BlockSpec((D, blk), lambda i: (0, i))],
        out_specs=pl.BlockSpec((blk, D), lambda i: (i, 0)),
    )(xt)


def _sc_gather(s, r, o, sp, op, e_table, r_table):
    mesh = plsc.VectorSubcoreMesh(core_axis_name="c", subcore_axis_name="s")

    @functools.partial(
        pl.kernel,
        out_type=jax.ShapeDtypeStruct((5 * B, D), jnp.float32),
        mesh=mesh,
        scratch_types=[
            pltpu.VMEM((RPT,), jnp.int32),
            pltpu.VMEM((RPT, D), jnp.float32),
            pltpu.SemaphoreType.DMA,
            pltpu.SemaphoreType.DMA,
        ],
        compiler_params=pltpu.CompilerParams(use_tc_tiling_on_sc=True),
    )
    def k(s_h, r_h, o_h, sp_h, op_h, e_h, rel_h, out_h, idx_v, rows_v,
          isem, gsem):
        wid = lax.axis_index("s") * NC + lax.axis_index("c")
        base = wid * RPT
        for slot, (idx_h, tbl_h) in enumerate(
            [(s_h, e_h), (r_h, rel_h), (o_h, e_h), (sp_h, e_h), (op_h, e_h)]
        ):
            pltpu.async_copy(idx_h.at[pl.ds(base, RPT)], idx_v, isem).wait()

            @pl.loop(0, RPT, step=16)
            def _fire(g):
                vec = idx_v[pl.ds(g, 16)]
                for l in range(16):
                    pltpu.async_copy(
                        tbl_h.at[pl.ds(vec[l], 1)], rows_v.at[pl.ds(g + l, 1)],
                        gsem,
                    )

            @pl.loop(0, RPT)
            def _drain(j):
                pltpu.make_async_copy(
                    tbl_h.at[pl.ds(0, 1)], rows_v.at[pl.ds(j, 1)], gsem
                ).wait()

            pltpu.sync_copy(rows_v, out_h.at[pl.ds(slot * B + base, RPT)])

    return k(s, r, o, sp, op, e_table, r_table)


def _tc_normalize(rows):
    n_rows = rows.shape[0]
    blk = 8192

    def body(x_ref, o_ref):
        x = x_ref[...]
        ss = jnp.sum(x * x, axis=1, keepdims=True)
        norm = jnp.sqrt(ss)
        o_ref[...] = x / jnp.maximum(norm, 1e-12)

    return pl.pallas_call(
        body,
        out_shape=jax.ShapeDtypeStruct((n_rows, D), jnp.float32),
        grid=(n_rows // blk,),
        in_specs=[pl.BlockSpec((blk, D), lambda i: (i, 0))],
        out_specs=pl.BlockSpec((blk, D), lambda i: (i, 0)),
    )(rows)


def kernel(s, r, o, sp, op, e_table, r_table):
    s = s.astype(jnp.int32)
    r = r.astype(jnp.int32)
    o = o.astype(jnp.int32)
    sp = sp.astype(jnp.int32)
    op = op.astype(jnp.int32)
    e_rowmajor = _tc_transpose(jnp.swapaxes(e_table, 0, 1))
    rows = _sc_gather(s, r, o, sp, op, e_rowmajor, r_table)
    out = _tc_normalize(rows)
    return out.reshape(5, B, D)


# speedup vs baseline: 1.2909x; 1.0826x over previous
import functools

import jax
import jax.numpy as jnp
from jax import lax
from jax.experimental import pallas as pl
from jax.experimental.pallas import tpu as pltpu
from jax.experimental.pallas import tpu_sc as plsc

NUM_E = 1000000
NUM_R = 1000
D = 64
B = 16384

NC = 2
NS = 16
NW = NC * NS
RPT = B // NW


def _tc_normalize_transpose(xt, blk):
    n = xt.shape[1]

    def body(x_ref, o_ref):
        x = x_ref[...]
        ss = jnp.sum(x * x, axis=0, keepdims=True)
        norm = jnp.sqrt(ss)
        y = x / jnp.maximum(norm, 1e-12)
        o_ref[...] = y.T

    return pl.pallas_call(
        body,
        out_shape=jax.ShapeDtypeStruct((n, D), jnp.float32),
        grid=(pl.cdiv(n, blk),),
        in_specs=[pl.BlockSpec((D, blk), lambda i: (0, i))],
        out_specs=pl.BlockSpec((blk, D), lambda i: (i, 0)),
    )(xt)


def _sc_gather(s, r, o, sp, op, e_norm, r_norm):
    mesh = plsc.VectorSubcoreMesh(core_axis_name="c", subcore_axis_name="s")

    @functools.partial(
        pl.kernel,
        out_type=jax.ShapeDtypeStruct((5 * B, D), jnp.float32),
        mesh=mesh,
        scratch_types=[
            pltpu.VMEM((RPT,), jnp.int32),
            pltpu.VMEM((RPT, D), jnp.float32),
            pltpu.SemaphoreType.DMA,
            pltpu.SemaphoreType.DMA,
        ],
        compiler_params=pltpu.CompilerParams(use_tc_tiling_on_sc=True),
    )
    def k(s_h, r_h, o_h, sp_h, op_h, e_h, rel_h, out_h, idx_v, rows_v,
          isem, gsem):
        wid = lax.axis_index("s") * NC + lax.axis_index("c")
        base = wid * RPT
        for slot, (idx_h, tbl_h) in enumerate(
            [(s_h, e_h), (r_h, rel_h), (o_h, e_h), (sp_h, e_h), (op_h, e_h)]
        ):
            pltpu.async_copy(idx_h.at[pl.ds(base, RPT)], idx_v, isem).wait()

            @pl.loop(0, RPT, step=16)
            def _fire(g):
                vec = idx_v[pl.ds(g, 16)]
                for l in range(16):
                    pltpu.async_copy(
                        tbl_h.at[pl.ds(vec[l], 1)], rows_v.at[pl.ds(g + l, 1)],
                        gsem,
                    )

            @pl.loop(0, RPT)
            def _drain(j):
                pltpu.make_async_copy(
                    tbl_h.at[pl.ds(0, 1)], rows_v.at[pl.ds(j, 1)], gsem
                ).wait()

            pltpu.sync_copy(rows_v, out_h.at[pl.ds(slot * B + base, RPT)])

    return k(s, r, o, sp, op, e_norm, r_norm)


def kernel(s, r, o, sp, op, e_table, r_table):
    s = s.astype(jnp.int32)
    r = r.astype(jnp.int32)
    o = o.astype(jnp.int32)
    sp = sp.astype(jnp.int32)
    op = op.astype(jnp.int32)
    e_norm = _tc_normalize_transpose(jnp.swapaxes(e_table, 0, 1), blk=8192)
    r_norm = _tc_normalize_transpose(jnp.swapaxes(r_table, 0, 1), blk=1024)
    rows = _sc_gather(s, r, o, sp, op, e_norm, r_norm)
    return rows.reshape(5, B, D)
